# baseline (speedup 1.0000x reference)
import functools

import numpy as np
import jax
import jax.numpy as jnp
from jax import lax
from jax.experimental import pallas as pl
from jax.experimental.pallas import tpu as pltpu
from jax.experimental.pallas import tpu_sc as plsc

N = 10000
E = 320000
D = 128
WPAD = 144
NC, NS = 2, 16
NW = NC * NS
EPW = E // NW
GB = 32
NCHT = E // GB
SUB = GB // 16
RPW = N // NS
ZB = 25
NEG = 0.2

_MESH = plsc.VectorSubcoreMesh(core_axis_name="c", subcore_axis_name="s")
_SC_PARAMS = pltpu.CompilerParams(use_tc_tiling_on_sc=False,
                                  needs_layout_passes=False,
                                  disable_bounds_checks=True)


def _mm_body(x_ref, w_ref, o_ref):
    o_ref[...] = jnp.dot(x_ref[...], w_ref[...],
                         preferred_element_type=jnp.float32)


def _matmul(x, w, bn=2000):
    n, k = x.shape
    m = w.shape[1]
    return pl.pallas_call(
        _mm_body,
        grid=(n // bn,),
        in_specs=[pl.BlockSpec((bn, k), lambda i: (i, 0)),
                  pl.BlockSpec((k, m), lambda i: (0, 0))],
        out_specs=pl.BlockSpec((bn, m), lambda i: (i, 0)),
        out_shape=jax.ShapeDtypeStruct((n, m), jnp.float32),
    )(x, w)


def _fused_body(heads, feat, src, dst, attn2, zeros, out,
                idxs_v, idxd_v, sidx_v, fsb, fdb, msgb, attn_v,
                gsem, asem, acc):
    hid = D // heads
    c = lax.axis_index("c")
    s = lax.axis_index("s")
    wid = c * NS + s
    nch = NCHT // NW + jnp.where(wid < NCHT % NW, 1, 0)

    pltpu.sync_copy(attn2, attn_v)

    pltpu.sync_copy(zeros.at[pl.ds(s * RPW, RPW)],
                    acc.at[pl.ds(s * RPW, RPW)])
    plsc.subcore_barrier()

    iota16 = lax.iota(jnp.int32, 16)
    zero16 = jnp.zeros((16,), jnp.float32)
    zidx16 = jnp.zeros((16,), jnp.int32)
    oidx16 = jnp.ones((16,), jnp.int32)
    rots = [jnp.bitwise_and(iota16 + dd, 15) for dd in range(16)]

    for row in range(2 * GB):
        msgb[row, pl.ds(D, 16)] = zero16

    def cbase(i):
        return (wid + NW * i) * GB

    def gstage(i):
        slot = lax.rem(i, 2)
        base = cbase(i)
        pltpu.sync_copy(src.at[pl.ds(base, GB)], idxs_v.at[slot])
        pltpu.sync_copy(dst.at[pl.ds(base, GB)], idxd_v.at[slot])
        pltpu.async_copy(feat.at[idxs_v.at[slot]],
                         fsb.at[pl.ds(slot * GB, GB)], gsem)
        pltpu.async_copy(feat.at[idxd_v.at[slot]],
                         fdb.at[pl.ds(slot * GB, GB)], gsem)

    def gwait(i):
        slot = lax.rem(i, 2)
        pltpu.make_async_copy(feat.at[idxs_v.at[slot]],
                              fsb.at[pl.ds(slot * GB, GB)], gsem).wait()
        pltpu.make_async_copy(feat.at[idxd_v.at[slot]],
                              fdb.at[pl.ds(slot * GB, GB)], gsem).wait()

    def add_start(i):
        slot = lax.rem(i, 2)
        pltpu.async_copy(msgb.at[pl.ds(slot * GB, GB)],
                         acc.at[sidx_v.at[slot]], asem, add=True)

    def add_wait(i):
        slot = lax.rem(i, 2)
        pltpu.make_async_copy(msgb.at[pl.ds(slot * GB, GB)],
                              acc.at[sidx_v.at[slot]], asem).wait()

    def compute(slot):
        def sub(r, carry):
            rows = slot * GB + r * 16 + iota16
            pvals = []
            for h in range(heads):
                def blk1(bi, zc):
                    dbase = h * hid + bi * 16
                    bidx = dbase + iota16
                    a06v = plsc.load_gather(attn_v, [zidx16, bidx])
                    a04v = plsc.load_gather(attn_v, [oidx16, bidx])
                    z = list(zc)
                    for dd in range(16):
                        cd = dbase + rots[dd]
                        a = plsc.load_gather(fsb, [rows, cd])
                        b = plsc.load_gather(fdb, [rows, cd])
                        a06 = a06v.at[rots[dd]].get(
                            mode="promise_in_bounds")
                        a04 = a04v.at[rots[dd]].get(
                            mode="promise_in_bounds")
                        sv = a + b
                        k = dd % 4
                        z[k] = z[k] + a06 * sv
                        k2 = 4 + dd % 4
                        z[k2] = z[k2] + a04 * jnp.abs(sv)
                    return tuple(z)

                zt = lax.fori_loop(0, hid // 16, blk1, (zero16,) * 8)
                zh = ((zt[0] + zt[1]) + (zt[2] + zt[3])
                      + (zt[4] + zt[5]) + (zt[6] + zt[7]))
                pvals.append(jnp.exp(zh))
            for h in range(heads):
                p = pvals[h]

                def blk2(bi, carry2):
                    dbase = h * hid + bi * 16
                    for dd in range(16):
                        cd = dbase + rots[dd]
                        a = plsc.load_gather(fsb, [rows, cd])
                        plsc.store_scatter(msgb, [rows, cd], a * p)
                    return carry2

                lax.fori_loop(0, hid // 16, blk2, 0)
            for j in range(heads):
                cj = jnp.full((16,), D + j, jnp.int32)
                plsc.store_scatter(msgb, [rows, cj], pvals[j])
            return carry

        lax.fori_loop(0, SUB, sub, 0)

    gstage(0)

    def step(i, carry):
        slot = lax.rem(i, 2)

        @pl.when(i + 1 < nch)
        def _():
            gstage(i + 1)

        gwait(i)
        compute(slot)
        for k in range(GB // 16):
            sidx_v[slot, pl.ds(k * 16, 16)] = idxd_v[slot, pl.ds(k * 16, 16)]

        @pl.when(i >= 1)
        def _():
            add_wait(i - 1)

        add_start(i)
        return carry

    lax.fori_loop(0, nch, step, 0)
    add_wait(nch - 1)
    plsc.subcore_barrier()
    pltpu.sync_copy(acc.at[pl.ds(s * RPW, RPW)],
                    out.at[c, pl.ds(s * RPW, RPW)])


def _make_fused(heads):
    return pl.kernel(
        functools.partial(_fused_body, heads),
        out_type=jax.ShapeDtypeStruct((NC, N, WPAD), jnp.float32),
        mesh=_MESH,
        scratch_types=[pltpu.VMEM((2, GB), jnp.int32),
                       pltpu.VMEM((2, GB), jnp.int32),
                       pltpu.VMEM((2, GB), jnp.int32),
                       pltpu.VMEM((2 * GB, D), jnp.float32),
                       pltpu.VMEM((2 * GB, D), jnp.float32),
                       pltpu.VMEM((2 * GB, WPAD), jnp.float32),
                       pltpu.VMEM((2, D), jnp.float32),
                       pltpu.SemaphoreType.DMA,
                       pltpu.SemaphoreType.DMA,
                       pltpu.VMEM_SHARED((N, WPAD), jnp.float32)],
        compiler_params=_SC_PARAMS,
    )


_fused4 = _make_fused(4)
_fused1 = _make_fused(1)


def _fin_mm_body(acc_ref, emat_ref, b_ref, w_ref, o_ref):
    a = acc_ref[0] + acc_ref[1]
    den = jnp.dot(a, emat_ref[...], preferred_element_type=jnp.float32)
    o = a[:, :D] / (den + 1e-9) + b_ref[0][None, :]
    o = jnp.where(o > 0, o, jnp.exp(o) - 1.0)
    o_ref[...] = jnp.dot(o, w_ref[...], preferred_element_type=jnp.float32)


def _fin_body(acc_ref, emat_ref, b_ref, o_ref):
    a = acc_ref[0] + acc_ref[1]
    den = jnp.dot(a, emat_ref[...], preferred_element_type=jnp.float32)
    o_ref[...] = a[:, :D] / (den + 1e-9) + b_ref[0][None, :]


def _finalize(acc, emat, b8, w=None, bn=2000):
    in_specs = [pl.BlockSpec((NC, bn, WPAD), lambda i: (0, i, 0)),
                pl.BlockSpec((WPAD, D), lambda i: (0, 0)),
                pl.BlockSpec((8, D), lambda i: (0, 0))]
    args = [acc, emat, b8]
    body = _fin_body
    if w is not None:
        in_specs.append(pl.BlockSpec((D, D), lambda i: (0, 0)))
        args.append(w)
        body = _fin_mm_body
    return pl.pallas_call(
        body,
        grid=(N // bn,),
        in_specs=in_specs,
        out_specs=pl.BlockSpec((bn, D), lambda i: (i, 0)),
        out_shape=jax.ShapeDtypeStruct((N, D), jnp.float32),
    )(*args)


def _emat_np(heads, hid):
    emat = np.zeros((WPAD, D), np.float32)
    for h in range(heads):
        emat[D + h, h * hid:(h + 1) * hid] = 1.0
    return emat


_E0 = _emat_np(4, 32)
_E1 = _emat_np(1, 128)


def _bcast8(v):
    return jnp.broadcast_to(v.reshape(1, D), (8, D))


def kernel(h, edge_index, W0, attn0, b0, W1, attn1, b1):
    src = edge_index[0]
    dst = edge_index[1]

    def layer(feat, attn, fused, emat, b, w_next):
        af = attn.reshape(-1)
        attn2 = jnp.stack([(1.0 + NEG) / 2.0 * af, (1.0 - NEG) / 2.0 * af])
        acc = fused(feat, src, dst, attn2,
                    jnp.zeros((N, WPAD), jnp.float32))
        return _finalize(acc, jnp.asarray(emat), _bcast8(b), w_next)

    feat0 = _matmul(h, W0)
    feat1 = layer(feat0, attn0, _fused4, _E0, b0, W1)
    out = layer(feat1, attn1, _fused1, _E1, b1, None)
    return out


# speedup vs baseline: 1.2402x; 1.2402x over previous
import functools

import numpy as np
import jax
import jax.numpy as jnp
from jax import lax
from jax.experimental import pallas as pl
from jax.experimental.pallas import tpu as pltpu
from jax.experimental.pallas import tpu_sc as plsc

N = 10000
E = 320000
D = 128
WPAD = 144
NC, NS = 2, 16
NW = NC * NS
EPW = E // NW
GB = 32
NCHT = E // GB
SUB = GB // 16
RPW = N // NS
ZB = 25
NEG = 0.2

_MESH = plsc.VectorSubcoreMesh(core_axis_name="c", subcore_axis_name="s")
_SC_PARAMS = pltpu.CompilerParams(use_tc_tiling_on_sc=False,
                                  needs_layout_passes=False,
                                  disable_bounds_checks=True)


def _mm_body(x_ref, w_ref, o_ref):
    o_ref[...] = jnp.dot(x_ref[...], w_ref[...],
                         preferred_element_type=jnp.float32)


def _matmul(x, w, bn=2000):
    n, k = x.shape
    m = w.shape[1]
    return pl.pallas_call(
        _mm_body,
        grid=(n // bn,),
        in_specs=[pl.BlockSpec((bn, k), lambda i: (i, 0)),
                  pl.BlockSpec((k, m), lambda i: (0, 0))],
        out_specs=pl.BlockSpec((bn, m), lambda i: (i, 0)),
        out_shape=jax.ShapeDtypeStruct((n, m), jnp.float32),
    )(x, w)


def _fused_body(heads, feat, src, dst, attn2, zeros, out,
                idxs_v, idxd_v, sidx_v, fsb, fdb, msgb, attn_v,
                gsem, asem, acc):
    hid = D // heads
    c = lax.axis_index("c")
    s = lax.axis_index("s")
    wid = c * NS + s
    nch = NCHT // NW + jnp.where(wid < NCHT % NW, 1, 0)

    pltpu.sync_copy(attn2, attn_v)

    pltpu.sync_copy(zeros.at[pl.ds(s * RPW, RPW)],
                    acc.at[pl.ds(s * RPW, RPW)])
    plsc.subcore_barrier()

    iota16 = lax.iota(jnp.int32, 16)
    zero16 = jnp.zeros((16,), jnp.float32)
    zidx16 = jnp.zeros((16,), jnp.int32)
    oidx16 = jnp.ones((16,), jnp.int32)
    rots = [jnp.bitwise_and(iota16 + dd, 15) for dd in range(16)]

    for row in range(2 * GB):
        msgb[row, pl.ds(D, 16)] = zero16

    def cbase(i):
        return (wid + NW * i) * GB

    def gstage(i):
        slot = lax.rem(i, 2)
        base = cbase(i)
        pltpu.sync_copy(src.at[pl.ds(base, GB)], idxs_v.at[slot])
        pltpu.sync_copy(dst.at[pl.ds(base, GB)], idxd_v.at[slot])
        pltpu.async_copy(feat.at[idxs_v.at[slot]],
                         fsb.at[pl.ds(slot * GB, GB)], gsem)
        pltpu.async_copy(feat.at[idxd_v.at[slot]],
                         fdb.at[pl.ds(slot * GB, GB)], gsem)

    def gwait(i):
        slot = lax.rem(i, 2)
        pltpu.make_async_copy(feat.at[idxs_v.at[slot]],
                              fsb.at[pl.ds(slot * GB, GB)], gsem).wait()
        pltpu.make_async_copy(feat.at[idxd_v.at[slot]],
                              fdb.at[pl.ds(slot * GB, GB)], gsem).wait()

    def add_start(i):
        slot = lax.rem(i, 2)
        pltpu.async_copy(msgb.at[pl.ds(slot * GB, GB)],
                         acc.at[sidx_v.at[slot]], asem, add=True)

    def add_wait(i):
        slot = lax.rem(i, 2)
        pltpu.make_async_copy(msgb.at[pl.ds(slot * GB, GB)],
                              acc.at[sidx_v.at[slot]], asem).wait()

    def compute(slot):
        def sub(r, carry):
            rows = slot * GB + r * 16 + iota16
            pvals = []
            for h in range(heads):
                def blk1(bi, zc):
                    zs, za = zc
                    dbase = h * hid + bi * 16
                    for dd in range(16):
                        cd = dbase + rots[dd]
                        a = plsc.load_gather(fsb, [rows, cd])
                        b = plsc.load_gather(fdb, [rows, cd])
                        a06 = plsc.load_gather(attn_v, [zidx16, cd])
                        a04 = plsc.load_gather(attn_v, [oidx16, cd])
                        sv = a + b
                        zs = zs + a06 * sv
                        za = za + a04 * jnp.abs(sv)
                    return zs, za

                zs, za = lax.fori_loop(0, hid // 16, blk1,
                                       (zero16, zero16))
                pvals.append(jnp.exp(zs + za))
            for h in range(heads):
                p = pvals[h]

                def blk2(bi, carry2):
                    dbase = h * hid + bi * 16
                    vals = []
                    for dd in range(16):
                        cd = dbase + rots[dd]
                        vals.append(plsc.load_gather(fsb, [rows, cd]) * p)
                    for dd in range(16):
                        cd = dbase + rots[dd]
                        plsc.store_scatter(msgb, [rows, cd], vals[dd])
                    return carry2

                lax.fori_loop(0, hid // 16, blk2, 0)
            for j in range(heads):
                cj = jnp.full((16,), D + j, jnp.int32)
                plsc.store_scatter(msgb, [rows, cj], pvals[j])
            return carry

        lax.fori_loop(0, SUB, sub, 0)

    gstage(0)

    def step(i, carry):
        slot = lax.rem(i, 2)

        @pl.when(i + 1 < nch)
        def _():
            gstage(i + 1)

        gwait(i)
        compute(slot)
        for k in range(GB // 16):
            sidx_v[slot, pl.ds(k * 16, 16)] = idxd_v[slot, pl.ds(k * 16, 16)]

        @pl.when(i >= 1)
        def _():
            add_wait(i - 1)

        add_start(i)
        return carry

    lax.fori_loop(0, nch, step, 0)
    add_wait(nch - 1)
    plsc.subcore_barrier()
    pltpu.sync_copy(acc.at[pl.ds(s * RPW, RPW)],
                    out.at[c, pl.ds(s * RPW, RPW)])


def _make_fused(heads):
    return pl.kernel(
        functools.partial(_fused_body, heads),
        out_type=jax.ShapeDtypeStruct((NC, N, WPAD), jnp.float32),
        mesh=_MESH,
        scratch_types=[pltpu.VMEM((2, GB), jnp.int32),
                       pltpu.VMEM((2, GB), jnp.int32),
                       pltpu.VMEM((2, GB), jnp.int32),
                       pltpu.VMEM((2 * GB, D), jnp.float32),
                       pltpu.VMEM((2 * GB, D), jnp.float32),
                       pltpu.VMEM((2 * GB, WPAD), jnp.float32),
                       pltpu.VMEM((2, D), jnp.float32),
                       pltpu.SemaphoreType.DMA,
                       pltpu.SemaphoreType.DMA,
                       pltpu.VMEM_SHARED((N, WPAD), jnp.float32)],
        compiler_params=_SC_PARAMS,
    )


_fused4 = _make_fused(4)
_fused1 = _make_fused(1)


def _fin_mm_body(acc_ref, emat_ref, b_ref, w_ref, o_ref):
    a = acc_ref[0] + acc_ref[1]
    den = jnp.dot(a, emat_ref[...], preferred_element_type=jnp.float32)
    o = a[:, :D] / (den + 1e-9) + b_ref[0][None, :]
    o = jnp.where(o > 0, o, jnp.exp(o) - 1.0)
    o_ref[...] = jnp.dot(o, w_ref[...], preferred_element_type=jnp.float32)


def _fin_body(acc_ref, emat_ref, b_ref, o_ref):
    a = acc_ref[0] + acc_ref[1]
    den = jnp.dot(a, emat_ref[...], preferred_element_type=jnp.float32)
    o_ref[...] = a[:, :D] / (den + 1e-9) + b_ref[0][None, :]


def _finalize(acc, emat, b8, w=None, bn=2000):
    in_specs = [pl.BlockSpec((NC, bn, WPAD), lambda i: (0, i, 0)),
                pl.BlockSpec((WPAD, D), lambda i: (0, 0)),
                pl.BlockSpec((8, D), lambda i: (0, 0))]
    args = [acc, emat, b8]
    body = _fin_body
    if w is not None:
        in_specs.append(pl.BlockSpec((D, D), lambda i: (0, 0)))
        args.append(w)
        body = _fin_mm_body
    return pl.pallas_call(
        body,
        grid=(N // bn,),
        in_specs=in_specs,
        out_specs=pl.BlockSpec((bn, D), lambda i: (i, 0)),
        out_shape=jax.ShapeDtypeStruct((N, D), jnp.float32),
    )(*args)


def _emat_np(heads, hid):
    emat = np.zeros((WPAD, D), np.float32)
    for h in range(heads):
        emat[D + h, h * hid:(h + 1) * hid] = 1.0
    return emat


_E0 = _emat_np(4, 32)
_E1 = _emat_np(1, 128)


def _bcast8(v):
    return jnp.broadcast_to(v.reshape(1, D), (8, D))


def kernel(h, edge_index, W0, attn0, b0, W1, attn1, b1):
    src = edge_index[0]
    dst = edge_index[1]

    def layer(feat, attn, fused, emat, b, w_next):
        af = attn.reshape(-1)
        attn2 = jnp.stack([(1.0 + NEG) / 2.0 * af, (1.0 - NEG) / 2.0 * af])
        acc = fused(feat, src, dst, attn2,
                    jnp.zeros((N, WPAD), jnp.float32))
        return _finalize(acc, jnp.asarray(emat), _bcast8(b), w_next)

    feat0 = _matmul(h, W0)
    feat1 = layer(feat0, attn0, _fused4, _E0, b0, W1)
    out = layer(feat1, attn1, _fused1, _E1, b1, None)
    return out


# speedup vs baseline: 1.3002x; 1.0484x over previous
import functools

import numpy as np
import jax
import jax.numpy as jnp
from jax import lax
from jax.experimental import pallas as pl
from jax.experimental.pallas import tpu as pltpu
from jax.experimental.pallas import tpu_sc as plsc

N = 10000
E = 320000
D = 128
WPAD = 144
NC, NS = 2, 16
NW = NC * NS
EPW = E // NW
GB = 32
NCHT = E // GB
SUB = GB // 16
RPW = N // NS
ZB = 25
NEG = 0.2

_MESH = plsc.VectorSubcoreMesh(core_axis_name="c", subcore_axis_name="s")
_SC_PARAMS = pltpu.CompilerParams(use_tc_tiling_on_sc=False,
                                  needs_layout_passes=False,
                                  disable_bounds_checks=True)


def _mm_body(x_ref, w_ref, o_ref):
    o_ref[...] = jnp.dot(x_ref[...], w_ref[...],
                         preferred_element_type=jnp.float32)


def _matmul(x, w, bn=2000):
    n, k = x.shape
    m = w.shape[1]
    return pl.pallas_call(
        _mm_body,
        grid=(n // bn,),
        in_specs=[pl.BlockSpec((bn, k), lambda i: (i, 0)),
                  pl.BlockSpec((k, m), lambda i: (0, 0))],
        out_specs=pl.BlockSpec((bn, m), lambda i: (i, 0)),
        out_shape=jax.ShapeDtypeStruct((n, m), jnp.float32),
    )(x, w)


def _fused_body(heads, feat, src, dst, attn2, zeros, out,
                idxs_v, idxd_v, sidx_v, fsb, fdb, msgb, attn_v,
                gsem, asem, acc):
    hid = D // heads
    c = lax.axis_index("c")
    s = lax.axis_index("s")
    wid = c * NS + s
    nch = NCHT // NW + jnp.where(wid < NCHT % NW, 1, 0)

    pltpu.sync_copy(attn2, attn_v)

    pltpu.sync_copy(zeros.at[pl.ds(s * RPW, RPW)],
                    acc.at[pl.ds(s * RPW, RPW)])
    plsc.subcore_barrier()

    iota16 = lax.iota(jnp.int32, 16)
    zero16 = jnp.zeros((16,), jnp.float32)
    zidx16 = jnp.zeros((16,), jnp.int32)
    oidx16 = jnp.ones((16,), jnp.int32)
    rots = [jnp.bitwise_and(iota16 + dd, 15) for dd in range(16)]

    for row in range(2 * GB):
        msgb[row, pl.ds(D, 16)] = zero16

    def cbase(i):
        return (wid + NW * i) * GB

    def gstage(i):
        slot = lax.rem(i, 2)
        base = cbase(i)
        pltpu.sync_copy(src.at[pl.ds(base, GB)], idxs_v.at[slot])
        pltpu.sync_copy(dst.at[pl.ds(base, GB)], idxd_v.at[slot])
        pltpu.async_copy(feat.at[idxs_v.at[slot]],
                         fsb.at[pl.ds(slot * GB, GB)], gsem)
        pltpu.async_copy(feat.at[idxd_v.at[slot]],
                         fdb.at[pl.ds(slot * GB, GB)], gsem)

    def gwait(i):
        slot = lax.rem(i, 2)
        pltpu.make_async_copy(feat.at[idxs_v.at[slot]],
                              fsb.at[pl.ds(slot * GB, GB)], gsem).wait()
        pltpu.make_async_copy(feat.at[idxd_v.at[slot]],
                              fdb.at[pl.ds(slot * GB, GB)], gsem).wait()

    def add_start(i):
        slot = lax.rem(i, 2)
        pltpu.async_copy(msgb.at[pl.ds(slot * GB, GB)],
                         acc.at[sidx_v.at[slot]], asem, add=True)

    def add_wait(i):
        slot = lax.rem(i, 2)
        pltpu.make_async_copy(msgb.at[pl.ds(slot * GB, GB)],
                              acc.at[sidx_v.at[slot]], asem).wait()

    def compute(slot):
        def sub(r, carry):
            rows = slot * GB + r * 16 + iota16
            pvals = []
            for h in range(heads):
                def blk1(bi, zc):
                    zs, za = zc
                    dbase = h * hid + bi * 16
                    for g in range(2):
                        svs = []
                        for dd in range(g * 8, g * 8 + 8):
                            cd = dbase + rots[dd]
                            a = plsc.load_gather(fsb, [rows, cd])
                            b = plsc.load_gather(fdb, [rows, cd])
                            svs.append(a + b)
                        for k, dd in enumerate(range(g * 8, g * 8 + 8)):
                            cd = dbase + rots[dd]
                            a06 = plsc.load_gather(attn_v, [zidx16, cd])
                            a04 = plsc.load_gather(attn_v, [oidx16, cd])
                            zs = zs + a06 * svs[k]
                            za = za + a04 * jnp.abs(svs[k])
                    return zs, za

                zs, za = lax.fori_loop(0, hid // 16, blk1,
                                       (zero16, zero16))
                pvals.append(jnp.exp(zs + za))
            for h in range(heads):
                p = pvals[h]

                def blk2(bi, carry2):
                    dbase = h * hid + bi * 16
                    vals = []
                    for dd in range(16):
                        cd = dbase + rots[dd]
                        vals.append(plsc.load_gather(fsb, [rows, cd]) * p)
                    for dd in range(16):
                        cd = dbase + rots[dd]
                        plsc.store_scatter(msgb, [rows, cd], vals[dd])
                    return carry2

                lax.fori_loop(0, hid // 16, blk2, 0)
            for j in range(heads):
                cj = jnp.full((16,), D + j, jnp.int32)
                plsc.store_scatter(msgb, [rows, cj], pvals[j])
            return carry

        lax.fori_loop(0, SUB, sub, 0)

    gstage(0)

    def step(i, carry):
        slot = lax.rem(i, 2)

        @pl.when(i + 1 < nch)
        def _():
            gstage(i + 1)

        gwait(i)
        compute(slot)
        for k in range(GB // 16):
            sidx_v[slot, pl.ds(k * 16, 16)] = idxd_v[slot, pl.ds(k * 16, 16)]

        @pl.when(i >= 1)
        def _():
            add_wait(i - 1)

        add_start(i)
        return carry

    lax.fori_loop(0, nch, step, 0)
    add_wait(nch - 1)
    plsc.subcore_barrier()
    pltpu.sync_copy(acc.at[pl.ds(s * RPW, RPW)],
                    out.at[c, pl.ds(s * RPW, RPW)])


def _make_fused(heads):
    return pl.kernel(
        functools.partial(_fused_body, heads),
        out_type=jax.ShapeDtypeStruct((NC, N, WPAD), jnp.float32),
        mesh=_MESH,
        scratch_types=[pltpu.VMEM((2, GB), jnp.int32),
                       pltpu.VMEM((2, GB), jnp.int32),
                       pltpu.VMEM((2, GB), jnp.int32),
                       pltpu.VMEM((2 * GB, D), jnp.float32),
                       pltpu.VMEM((2 * GB, D), jnp.float32),
                       pltpu.VMEM((2 * GB, WPAD), jnp.float32),
                       pltpu.VMEM((2, D), jnp.float32),
                       pltpu.SemaphoreType.DMA,
                       pltpu.SemaphoreType.DMA,
                       pltpu.VMEM_SHARED((N, WPAD), jnp.float32)],
        compiler_params=_SC_PARAMS,
    )


_fused4 = _make_fused(4)
_fused1 = _make_fused(1)


def _fin_mm_body(acc_ref, emat_ref, b_ref, w_ref, o_ref):
    a = acc_ref[0] + acc_ref[1]
    den = jnp.dot(a, emat_ref[...], preferred_element_type=jnp.float32)
    o = a[:, :D] / (den + 1e-9) + b_ref[0][None, :]
    o = jnp.where(o > 0, o, jnp.exp(o) - 1.0)
    o_ref[...] = jnp.dot(o, w_ref[...], preferred_element_type=jnp.float32)


def _fin_body(acc_ref, emat_ref, b_ref, o_ref):
    a = acc_ref[0] + acc_ref[1]
    den = jnp.dot(a, emat_ref[...], preferred_element_type=jnp.float32)
    o_ref[...] = a[:, :D] / (den + 1e-9) + b_ref[0][None, :]


def _finalize(acc, emat, b8, w=None, bn=2000):
    in_specs = [pl.BlockSpec((NC, bn, WPAD), lambda i: (0, i, 0)),
                pl.BlockSpec((WPAD, D), lambda i: (0, 0)),
                pl.BlockSpec((8, D), lambda i: (0, 0))]
    args = [acc, emat, b8]
    body = _fin_body
    if w is not None:
        in_specs.append(pl.BlockSpec((D, D), lambda i: (0, 0)))
        args.append(w)
        body = _fin_mm_body
    return pl.pallas_call(
        body,
        grid=(N // bn,),
        in_specs=in_specs,
        out_specs=pl.BlockSpec((bn, D), lambda i: (i, 0)),
        out_shape=jax.ShapeDtypeStruct((N, D), jnp.float32),
    )(*args)


def _emat_np(heads, hid):
    emat = np.zeros((WPAD, D), np.float32)
    for h in range(heads):
        emat[D + h, h * hid:(h + 1) * hid] = 1.0
    return emat


_E0 = _emat_np(4, 32)
_E1 = _emat_np(1, 128)


def _bcast8(v):
    return jnp.broadcast_to(v.reshape(1, D), (8, D))


def kernel(h, edge_index, W0, attn0, b0, W1, attn1, b1):
    src = edge_index[0]
    dst = edge_index[1]

    def layer(feat, attn, fused, emat, b, w_next):
        af = attn.reshape(-1)
        attn2 = jnp.stack([(1.0 + NEG) / 2.0 * af, (1.0 - NEG) / 2.0 * af])
        acc = fused(feat, src, dst, attn2,
                    jnp.zeros((N, WPAD), jnp.float32))
        return _finalize(acc, jnp.asarray(emat), _bcast8(b), w_next)

    feat0 = _matmul(h, W0)
    feat1 = layer(feat0, attn0, _fused4, _E0, b0, W1)
    out = layer(feat1, attn1, _fused1, _E1, b1, None)
    return out


# speedup vs baseline: 1.3167x; 1.0127x over previous
import functools

import numpy as np
import jax
import jax.numpy as jnp
from jax import lax
from jax.experimental import pallas as pl
from jax.experimental.pallas import tpu as pltpu
from jax.experimental.pallas import tpu_sc as plsc

N = 10000
E = 320000
D = 128
WPAD = 144
NC, NS = 2, 16
NW = NC * NS
EPW = E // NW
GB = 32
NCHT = E // GB
SUB = GB // 16
RPW = N // NS
ZB = 25
NEG = 0.2

_MESH = plsc.VectorSubcoreMesh(core_axis_name="c", subcore_axis_name="s")
_SC_PARAMS = pltpu.CompilerParams(use_tc_tiling_on_sc=False,
                                  needs_layout_passes=False,
                                  disable_bounds_checks=True)


def _mm_body(x_ref, w_ref, o_ref):
    o_ref[...] = jnp.dot(x_ref[...], w_ref[...],
                         preferred_element_type=jnp.float32)


def _matmul(x, w, bn=2000):
    n, k = x.shape
    m = w.shape[1]
    return pl.pallas_call(
        _mm_body,
        grid=(n // bn,),
        in_specs=[pl.BlockSpec((bn, k), lambda i: (i, 0)),
                  pl.BlockSpec((k, m), lambda i: (0, 0))],
        out_specs=pl.BlockSpec((bn, m), lambda i: (i, 0)),
        out_shape=jax.ShapeDtypeStruct((n, m), jnp.float32),
    )(x, w)


def _fused_body(heads, feat, src, dst, attn2, zeros, out,
                idxs_v, idxd_v, sidx_v, fsb, fdb, msgb, attn_v,
                gsem, asem, acc):
    hid = D // heads
    c = lax.axis_index("c")
    s = lax.axis_index("s")
    wid = c * NS + s
    nch = NCHT // NW + jnp.where(wid < NCHT % NW, 1, 0)

    pltpu.sync_copy(attn2, attn_v)

    pltpu.sync_copy(zeros.at[pl.ds(s * RPW, RPW)],
                    acc.at[pl.ds(s * RPW, RPW)])
    plsc.subcore_barrier()

    iota16 = lax.iota(jnp.int32, 16)
    zero16 = jnp.zeros((16,), jnp.float32)
    zidx16 = jnp.zeros((16,), jnp.int32)
    oidx16 = jnp.ones((16,), jnp.int32)
    rots = [jnp.bitwise_and(iota16 + dd, 15) for dd in range(16)]

    for row in range(2 * GB):
        msgb[row, pl.ds(D, 16)] = zero16

    def cbase(i):
        return (wid + NW * i) * GB

    def gstage(i):
        slot = lax.rem(i, 2)
        base = cbase(i)
        pltpu.sync_copy(src.at[pl.ds(base, GB)], idxs_v.at[slot])
        pltpu.sync_copy(dst.at[pl.ds(base, GB)], idxd_v.at[slot])
        pltpu.async_copy(feat.at[idxs_v.at[slot]],
                         fsb.at[pl.ds(slot * GB, GB)], gsem)
        pltpu.async_copy(feat.at[idxd_v.at[slot]],
                         fdb.at[pl.ds(slot * GB, GB)], gsem)

    def gwait(i):
        slot = lax.rem(i, 2)
        pltpu.make_async_copy(feat.at[idxs_v.at[slot]],
                              fsb.at[pl.ds(slot * GB, GB)], gsem).wait()
        pltpu.make_async_copy(feat.at[idxd_v.at[slot]],
                              fdb.at[pl.ds(slot * GB, GB)], gsem).wait()

    def add_start(i):
        slot = lax.rem(i, 2)
        pltpu.async_copy(msgb.at[pl.ds(slot * GB, GB)],
                         acc.at[sidx_v.at[slot]], asem, add=True)

    def add_wait(i):
        slot = lax.rem(i, 2)
        pltpu.make_async_copy(msgb.at[pl.ds(slot * GB, GB)],
                              acc.at[sidx_v.at[slot]], asem).wait()

    def compute(slot):
        def sub(r, carry):
            rows = slot * GB + r * 16 + iota16
            pvals = []
            for h in range(heads):
                def blk1(bi, zc):
                    zs, za = zc
                    dbase = h * hid + bi * 16
                    for g in range(2):
                        svs = []
                        ats = []
                        for dd in range(g * 8, g * 8 + 8):
                            cd = dbase + rots[dd]
                            a = plsc.load_gather(fsb, [rows, cd])
                            b = plsc.load_gather(fdb, [rows, cd])
                            a06 = plsc.load_gather(attn_v, [zidx16, cd])
                            a04 = plsc.load_gather(attn_v, [oidx16, cd])
                            svs.append(a + b)
                            ats.append((a06, a04))
                        for k in range(8):
                            zs = zs + ats[k][0] * svs[k]
                            za = za + ats[k][1] * jnp.abs(svs[k])
                    return zs, za

                zs, za = lax.fori_loop(0, hid // 16, blk1,
                                       (zero16, zero16))
                pvals.append(jnp.exp(zs + za))
            for h in range(heads):
                p = pvals[h]

                def blk2(bi, carry2):
                    dbase = h * hid + bi * 16
                    vals = []
                    for dd in range(16):
                        cd = dbase + rots[dd]
                        vals.append(plsc.load_gather(fsb, [rows, cd]) * p)
                    for dd in range(16):
                        cd = dbase + rots[dd]
                        plsc.store_scatter(msgb, [rows, cd], vals[dd])
                    return carry2

                lax.fori_loop(0, hid // 16, blk2, 0)
            for j in range(heads):
                cj = jnp.full((16,), D + j, jnp.int32)
                plsc.store_scatter(msgb, [rows, cj], pvals[j])
            return carry

        lax.fori_loop(0, SUB, sub, 0)

    gstage(0)

    def step(i, carry):
        slot = lax.rem(i, 2)

        @pl.when(i + 1 < nch)
        def _():
            gstage(i + 1)

        gwait(i)
        compute(slot)
        for k in range(GB // 16):
            sidx_v[slot, pl.ds(k * 16, 16)] = idxd_v[slot, pl.ds(k * 16, 16)]

        @pl.when(i >= 1)
        def _():
            add_wait(i - 1)

        add_start(i)
        return carry

    lax.fori_loop(0, nch, step, 0)
    add_wait(nch - 1)
    plsc.subcore_barrier()
    pltpu.sync_copy(acc.at[pl.ds(s * RPW, RPW)],
                    out.at[c, pl.ds(s * RPW, RPW)])


def _make_fused(heads):
    return pl.kernel(
        functools.partial(_fused_body, heads),
        out_type=jax.ShapeDtypeStruct((NC, N, WPAD), jnp.float32),
        mesh=_MESH,
        scratch_types=[pltpu.VMEM((2, GB), jnp.int32),
                       pltpu.VMEM((2, GB), jnp.int32),
                       pltpu.VMEM((2, GB), jnp.int32),
                       pltpu.VMEM((2 * GB, D), jnp.float32),
                       pltpu.VMEM((2 * GB, D), jnp.float32),
                       pltpu.VMEM((2 * GB, WPAD), jnp.float32),
                       pltpu.VMEM((2, D), jnp.float32),
                       pltpu.SemaphoreType.DMA,
                       pltpu.SemaphoreType.DMA,
                       pltpu.VMEM_SHARED((N, WPAD), jnp.float32)],
        compiler_params=_SC_PARAMS,
    )


_fused4 = _make_fused(4)
_fused1 = _make_fused(1)


def _fin_mm_body(acc_ref, emat_ref, b_ref, w_ref, o_ref):
    a = acc_ref[0] + acc_ref[1]
    den = jnp.dot(a, emat_ref[...], preferred_element_type=jnp.float32)
    o = a[:, :D] / (den + 1e-9) + b_ref[0][None, :]
    o = jnp.where(o > 0, o, jnp.exp(o) - 1.0)
    o_ref[...] = jnp.dot(o, w_ref[...], preferred_element_type=jnp.float32)


def _fin_body(acc_ref, emat_ref, b_ref, o_ref):
    a = acc_ref[0] + acc_ref[1]
    den = jnp.dot(a, emat_ref[...], preferred_element_type=jnp.float32)
    o_ref[...] = a[:, :D] / (den + 1e-9) + b_ref[0][None, :]


def _finalize(acc, emat, b8, w=None, bn=2000):
    in_specs = [pl.BlockSpec((NC, bn, WPAD), lambda i: (0, i, 0)),
                pl.BlockSpec((WPAD, D), lambda i: (0, 0)),
                pl.BlockSpec((8, D), lambda i: (0, 0))]
    args = [acc, emat, b8]
    body = _fin_body
    if w is not None:
        in_specs.append(pl.BlockSpec((D, D), lambda i: (0, 0)))
        args.append(w)
        body = _fin_mm_body
    return pl.pallas_call(
        body,
        grid=(N // bn,),
        in_specs=in_specs,
        out_specs=pl.BlockSpec((bn, D), lambda i: (i, 0)),
        out_shape=jax.ShapeDtypeStruct((N, D), jnp.float32),
    )(*args)


def _emat_np(heads, hid):
    emat = np.zeros((WPAD, D), np.float32)
    for h in range(heads):
        emat[D + h, h * hid:(h + 1) * hid] = 1.0
    return emat


_E0 = _emat_np(4, 32)
_E1 = _emat_np(1, 128)


def _bcast8(v):
    return jnp.broadcast_to(v.reshape(1, D), (8, D))


def kernel(h, edge_index, W0, attn0, b0, W1, attn1, b1):
    src = edge_index[0]
    dst = edge_index[1]

    def layer(feat, attn, fused, emat, b, w_next):
        af = attn.reshape(-1)
        attn2 = jnp.stack([(1.0 + NEG) / 2.0 * af, (1.0 - NEG) / 2.0 * af])
        acc = fused(feat, src, dst, attn2,
                    jnp.zeros((N, WPAD), jnp.float32))
        return _finalize(acc, jnp.asarray(emat), _bcast8(b), w_next)

    feat0 = _matmul(h, W0)
    feat1 = layer(feat0, attn0, _fused4, _E0, b0, W1)
    out = layer(feat1, attn1, _fused1, _E1, b1, None)
    return out
